# baseline (speedup 1.0000x reference)
import functools

import jax
import jax.numpy as jnp
from jax import lax
from jax.experimental import pallas as pl
from jax.experimental.pallas import tpu as pltpu
from jax.experimental.pallas import tpu_sc as plsc

N = 10000
E = 320000
D = 128
C = 40

WPAD = 48
DEGW = 16
NC = 2
NS = 16
NW = NC * NS
CH = 256
N_PAD = 10240
ROWS_PER_TILE = N_PAD // NS
E_PAD = ((E + NW * CH - 1) // (NW * CH)) * (NW * CH)
ITERS = E_PAD // (NW * CH)

_mesh = plsc.VectorSubcoreMesh(core_axis_name="c", subcore_axis_name="s")


@functools.partial(
    pl.kernel,
    mesh=_mesh,
    compiler_params=pltpu.CompilerParams(use_tc_tiling_on_sc=False),
    out_type=jax.ShapeDtypeStruct((NC, N_PAD, WPAD), jnp.float32),
    scratch_types=[
        pltpu.VMEM((ITERS, CH), jnp.int32),
        pltpu.VMEM((ITERS, CH), jnp.int32),
        pltpu.VMEM((CH, WPAD), jnp.float32),
        pltpu.VMEM_SHARED((N_PAD, WPAD), jnp.float32),
        pltpu.SemaphoreType.DMA,
    ],
)
def _sc_hop(table_hbm, src_hbm, dst_hbm, zeros_hbm, out_hbm,
            src_v, dst_v, rows_a, acc_sh, sem):
    c = lax.axis_index("c")
    s = lax.axis_index("s")
    wid = s * NC + c
    base = s * ROWS_PER_TILE
    pltpu.sync_copy(zeros_hbm, acc_sh.at[pl.ds(base, ROWS_PER_TILE)])
    pltpu.sync_copy(src_hbm.at[wid], src_v)
    pltpu.sync_copy(dst_hbm.at[wid], dst_v)
    plsc.subcore_barrier()

    def body(j, carry):
        pltpu.async_copy(table_hbm.at[src_v.at[j]], rows_a, sem).wait()
        pltpu.sync_copy(rows_a, acc_sh.at[dst_v.at[j]], add=True)
        return carry

    lax.fori_loop(0, ITERS, body, 0)
    plsc.subcore_barrier()
    pltpu.sync_copy(
        acc_sh.at[pl.ds(base, ROWS_PER_TILE)],
        out_hbm.at[c].at[pl.ds(base, ROWS_PER_TILE)],
    )


@functools.partial(
    pl.kernel,
    mesh=_mesh,
    compiler_params=pltpu.CompilerParams(use_tc_tiling_on_sc=False),
    out_type=jax.ShapeDtypeStruct((NC, N_PAD, DEGW), jnp.float32),
    scratch_types=[
        pltpu.VMEM((ITERS, CH), jnp.int32),
        pltpu.VMEM((ITERS, CH), jnp.int32),
        pltpu.VMEM((CH, DEGW), jnp.float32),
        pltpu.VMEM_SHARED((N_PAD, DEGW), jnp.float32),
        pltpu.SemaphoreType.DMA,
    ],
)
def _sc_deg(table_hbm, src_hbm, dst_hbm, zeros_hbm, out_hbm,
            src_v, dst_v, rows_a, acc_sh, sem):
    c = lax.axis_index("c")
    s = lax.axis_index("s")
    wid = s * NC + c
    base = s * ROWS_PER_TILE
    pltpu.sync_copy(zeros_hbm, acc_sh.at[pl.ds(base, ROWS_PER_TILE)])
    pltpu.sync_copy(src_hbm.at[wid], src_v)
    pltpu.sync_copy(dst_hbm.at[wid], dst_v)
    plsc.subcore_barrier()

    def body(j, carry):
        pltpu.async_copy(table_hbm.at[src_v.at[j]], rows_a, sem).wait()
        pltpu.sync_copy(rows_a, acc_sh.at[dst_v.at[j]], add=True)
        return carry

    lax.fori_loop(0, ITERS, body, 0)
    plsc.subcore_barrier()
    pltpu.sync_copy(
        acc_sh.at[pl.ds(base, ROWS_PER_TILE)],
        out_hbm.at[c].at[pl.ds(base, ROWS_PER_TILE)],
    )


_BLK = 1024


def _tc_mm_scale(x_pad, w_pad, degp):
    def body(x_ref, w_ref, dp_ref, t0_ref, dinv_ref, dinv2_ref):
        y0 = jnp.dot(x_ref[...], w_ref[...], preferred_element_type=jnp.float32)
        dp = dp_ref[...]
        deg = dp[0, :, 0:1] + dp[1, :, 0:1] + 1.0
        dinv = lax.rsqrt(deg)
        dinv_ref[...] = dinv
        dinv2_ref[...] = 1.0 / deg
        t0_ref[...] = y0 * dinv

    grid = N_PAD // _BLK
    return pl.pallas_call(
        body,
        grid=(grid,),
        in_specs=[
            pl.BlockSpec((_BLK, D), lambda i: (i, 0)),
            pl.BlockSpec((D, WPAD), lambda i: (0, 0)),
            pl.BlockSpec((NC, _BLK, DEGW), lambda i: (0, i, 0)),
        ],
        out_specs=[
            pl.BlockSpec((_BLK, WPAD), lambda i: (i, 0)),
            pl.BlockSpec((_BLK, 1), lambda i: (i, 0)),
            pl.BlockSpec((_BLK, 1), lambda i: (i, 0)),
        ],
        out_shape=[
            jax.ShapeDtypeStruct((N_PAD, WPAD), jnp.float32),
            jax.ShapeDtypeStruct((N_PAD, 1), jnp.float32),
            jax.ShapeDtypeStruct((N_PAD, 1), jnp.float32),
        ],
    )(x_pad, w_pad, degp)


def _tc_combine(up, t_prev, scale):
    def body(up_ref, t_ref, s_ref, o_ref):
        u = up_ref[...]
        o_ref[...] = (u[0] + u[1] + t_ref[...]) * s_ref[...]

    grid = N_PAD // _BLK
    return pl.pallas_call(
        body,
        grid=(grid,),
        in_specs=[
            pl.BlockSpec((NC, _BLK, WPAD), lambda i: (0, i, 0)),
            pl.BlockSpec((_BLK, WPAD), lambda i: (i, 0)),
            pl.BlockSpec((_BLK, 1), lambda i: (i, 0)),
        ],
        out_specs=pl.BlockSpec((_BLK, WPAD), lambda i: (i, 0)),
        out_shape=jax.ShapeDtypeStruct((N_PAD, WPAD), jnp.float32),
    )(up, t_prev, scale)


def kernel(x, edge_index, W):
    src = edge_index[0]
    dst = edge_index[1]
    pad = E_PAD - E
    src_r = jnp.concatenate([src, jnp.zeros((pad,), jnp.int32)]).reshape(NW, ITERS, CH)
    dst_r = jnp.concatenate([dst, jnp.full((pad,), N, jnp.int32)]).reshape(NW, ITERS, CH)

    x_pad = jnp.pad(x, ((0, N_PAD - N), (0, 0)))
    w_pad = jnp.pad(W, ((0, 0), (0, WPAD - C)))

    zeros_hop = jnp.zeros((ROWS_PER_TILE, WPAD), jnp.float32)
    zeros_deg = jnp.zeros((ROWS_PER_TILE, DEGW), jnp.float32)
    ones_table = jnp.ones((N_PAD, DEGW), jnp.float32)

    degp = _sc_deg(ones_table, src_r, dst_r, zeros_deg)
    t0, dinv, dinv2 = _tc_mm_scale(x_pad, w_pad, degp)
    u1p = _sc_hop(t0, src_r, dst_r, zeros_hop)
    t1 = _tc_combine(u1p, t0, dinv2)
    u2p = _sc_hop(t1, src_r, dst_r, zeros_hop)
    out = _tc_combine(u2p, t1, dinv)
    return out[:N, :C]


# speedup vs baseline: 1.0742x; 1.0742x over previous
import functools

import jax
import jax.numpy as jnp
from jax import lax
from jax.experimental import pallas as pl
from jax.experimental.pallas import tpu as pltpu
from jax.experimental.pallas import tpu_sc as plsc

N = 10000
E = 320000
D = 128
C = 40

WPAD = 48
DEGW = 16
NC = 2
NS = 16
NW = NC * NS
CH = 64
N_PAD = 10240
ROWS_PER_TILE = N_PAD // NS
E_PAD = ((E + NW * CH - 1) // (NW * CH)) * (NW * CH)
ITERS = E_PAD // (NW * CH)

_mesh = plsc.VectorSubcoreMesh(core_axis_name="c", subcore_axis_name="s")


@functools.partial(
    pl.kernel,
    mesh=_mesh,
    compiler_params=pltpu.CompilerParams(use_tc_tiling_on_sc=False),
    out_type=jax.ShapeDtypeStruct((NC, N_PAD, WPAD), jnp.float32),
    scratch_types=[
        pltpu.VMEM((ITERS, CH), jnp.int32),
        pltpu.VMEM((ITERS, CH), jnp.int32),
        pltpu.VMEM((CH, WPAD), jnp.float32),
        pltpu.VMEM_SHARED((N_PAD, WPAD), jnp.float32),
        pltpu.SemaphoreType.DMA,
    ],
)
def _sc_hop(table_hbm, src_hbm, dst_hbm, zeros_hbm, out_hbm,
            src_v, dst_v, rows_a, acc_sh, sem):
    c = lax.axis_index("c")
    s = lax.axis_index("s")
    wid = s * NC + c
    base = s * ROWS_PER_TILE
    pltpu.sync_copy(zeros_hbm, acc_sh.at[pl.ds(base, ROWS_PER_TILE)])
    pltpu.sync_copy(src_hbm.at[wid], src_v)
    pltpu.sync_copy(dst_hbm.at[wid], dst_v)
    plsc.subcore_barrier()

    def body(j, carry):
        pltpu.async_copy(table_hbm.at[src_v.at[j]], rows_a, sem).wait()
        pltpu.sync_copy(rows_a, acc_sh.at[dst_v.at[j]], add=True)
        return carry

    lax.fori_loop(0, ITERS, body, 0)
    plsc.subcore_barrier()
    pltpu.sync_copy(
        acc_sh.at[pl.ds(base, ROWS_PER_TILE)],
        out_hbm.at[c].at[pl.ds(base, ROWS_PER_TILE)],
    )


@functools.partial(
    pl.kernel,
    mesh=_mesh,
    compiler_params=pltpu.CompilerParams(use_tc_tiling_on_sc=False),
    out_type=jax.ShapeDtypeStruct((NC, N_PAD, DEGW), jnp.float32),
    scratch_types=[
        pltpu.VMEM((ITERS, CH), jnp.int32),
        pltpu.VMEM((ITERS, CH), jnp.int32),
        pltpu.VMEM((CH, DEGW), jnp.float32),
        pltpu.VMEM_SHARED((N_PAD, DEGW), jnp.float32),
        pltpu.SemaphoreType.DMA,
    ],
)
def _sc_deg(table_hbm, src_hbm, dst_hbm, zeros_hbm, out_hbm,
            src_v, dst_v, rows_a, acc_sh, sem):
    c = lax.axis_index("c")
    s = lax.axis_index("s")
    wid = s * NC + c
    base = s * ROWS_PER_TILE
    pltpu.sync_copy(zeros_hbm, acc_sh.at[pl.ds(base, ROWS_PER_TILE)])
    pltpu.sync_copy(src_hbm.at[wid], src_v)
    pltpu.sync_copy(dst_hbm.at[wid], dst_v)
    plsc.subcore_barrier()

    def body(j, carry):
        pltpu.async_copy(table_hbm.at[src_v.at[j]], rows_a, sem).wait()
        pltpu.sync_copy(rows_a, acc_sh.at[dst_v.at[j]], add=True)
        return carry

    lax.fori_loop(0, ITERS, body, 0)
    plsc.subcore_barrier()
    pltpu.sync_copy(
        acc_sh.at[pl.ds(base, ROWS_PER_TILE)],
        out_hbm.at[c].at[pl.ds(base, ROWS_PER_TILE)],
    )


_BLK = 1024


def _tc_mm_scale(x_pad, w_pad, degp):
    def body(x_ref, w_ref, dp_ref, t0_ref, dinv_ref, dinv2_ref):
        y0 = jnp.dot(x_ref[...], w_ref[...], preferred_element_type=jnp.float32)
        dp = dp_ref[...]
        deg = dp[0, :, 0:1] + dp[1, :, 0:1] + 1.0
        dinv = lax.rsqrt(deg)
        dinv_ref[...] = dinv
        dinv2_ref[...] = 1.0 / deg
        t0_ref[...] = y0 * dinv

    grid = N_PAD // _BLK
    return pl.pallas_call(
        body,
        grid=(grid,),
        in_specs=[
            pl.BlockSpec((_BLK, D), lambda i: (i, 0)),
            pl.BlockSpec((D, WPAD), lambda i: (0, 0)),
            pl.BlockSpec((NC, _BLK, DEGW), lambda i: (0, i, 0)),
        ],
        out_specs=[
            pl.BlockSpec((_BLK, WPAD), lambda i: (i, 0)),
            pl.BlockSpec((_BLK, 1), lambda i: (i, 0)),
            pl.BlockSpec((_BLK, 1), lambda i: (i, 0)),
        ],
        out_shape=[
            jax.ShapeDtypeStruct((N_PAD, WPAD), jnp.float32),
            jax.ShapeDtypeStruct((N_PAD, 1), jnp.float32),
            jax.ShapeDtypeStruct((N_PAD, 1), jnp.float32),
        ],
    )(x_pad, w_pad, degp)


def _tc_combine(up, t_prev, scale):
    def body(up_ref, t_ref, s_ref, o_ref):
        u = up_ref[...]
        o_ref[...] = (u[0] + u[1] + t_ref[...]) * s_ref[...]

    grid = N_PAD // _BLK
    return pl.pallas_call(
        body,
        grid=(grid,),
        in_specs=[
            pl.BlockSpec((NC, _BLK, WPAD), lambda i: (0, i, 0)),
            pl.BlockSpec((_BLK, WPAD), lambda i: (i, 0)),
            pl.BlockSpec((_BLK, 1), lambda i: (i, 0)),
        ],
        out_specs=pl.BlockSpec((_BLK, WPAD), lambda i: (i, 0)),
        out_shape=jax.ShapeDtypeStruct((N_PAD, WPAD), jnp.float32),
    )(up, t_prev, scale)


def kernel(x, edge_index, W):
    src = edge_index[0]
    dst = edge_index[1]
    pad = E_PAD - E
    src_r = jnp.concatenate([src, jnp.zeros((pad,), jnp.int32)]).reshape(NW, ITERS, CH)
    dst_r = jnp.concatenate([dst, jnp.full((pad,), N, jnp.int32)]).reshape(NW, ITERS, CH)

    x_pad = jnp.pad(x, ((0, N_PAD - N), (0, 0)))
    w_pad = jnp.pad(W, ((0, 0), (0, WPAD - C)))

    zeros_hop = jnp.zeros((ROWS_PER_TILE, WPAD), jnp.float32)
    zeros_deg = jnp.zeros((ROWS_PER_TILE, DEGW), jnp.float32)
    ones_table = jnp.ones((N_PAD, DEGW), jnp.float32)

    degp = _sc_deg(ones_table, src_r, dst_r, zeros_deg)
    t0, dinv, dinv2 = _tc_mm_scale(x_pad, w_pad, degp)
    u1p = _sc_hop(t0, src_r, dst_r, zeros_hop)
    t1 = _tc_combine(u1p, t0, dinv2)
    u2p = _sc_hop(t1, src_r, dst_r, zeros_hop)
    out = _tc_combine(u2p, t1, dinv)
    return out[:N, :C]


# speedup vs baseline: 1.2021x; 1.1191x over previous
import functools

import jax
import jax.numpy as jnp
from jax import lax
from jax.experimental import pallas as pl
from jax.experimental.pallas import tpu as pltpu
from jax.experimental.pallas import tpu_sc as plsc

N = 10000
E = 320000
D = 128
C = 40

WPAD = 48
DEGW = 16
NC = 2
NS = 16
NW = NC * NS
CH = 128
N_PAD = 10240
ROWS_PER_TILE = N_PAD // NS
E_PAD = ((E + NW * CH - 1) // (NW * CH)) * (NW * CH)
ITERS = E_PAD // (NW * CH)

_mesh = plsc.VectorSubcoreMesh(core_axis_name="c", subcore_axis_name="s")


@functools.partial(
    pl.kernel,
    mesh=_mesh,
    compiler_params=pltpu.CompilerParams(use_tc_tiling_on_sc=False),
    out_type=jax.ShapeDtypeStruct((NC, N_PAD, WPAD), jnp.float32),
    scratch_types=[
        pltpu.VMEM((ITERS, CH), jnp.int32),
        pltpu.VMEM((ITERS, CH), jnp.int32),
        pltpu.VMEM((CH, WPAD), jnp.float32),
        pltpu.VMEM_SHARED((N_PAD, WPAD), jnp.float32),
        pltpu.SemaphoreType.DMA,
    ],
)
def _sc_hop(table_hbm, src_hbm, dst_hbm, zeros_hbm, out_hbm,
            src_v, dst_v, rows_a, acc_sh, sem):
    c = lax.axis_index("c")
    s = lax.axis_index("s")
    wid = s * NC + c
    base = s * ROWS_PER_TILE
    pltpu.sync_copy(zeros_hbm, acc_sh.at[pl.ds(base, ROWS_PER_TILE)])
    pltpu.sync_copy(src_hbm.at[wid], src_v)
    pltpu.sync_copy(dst_hbm.at[wid], dst_v)
    plsc.subcore_barrier()

    def body(j, carry):
        pltpu.async_copy(table_hbm.at[src_v.at[j]], rows_a, sem).wait()
        pltpu.sync_copy(rows_a, acc_sh.at[dst_v.at[j]], add=True)
        return carry

    lax.fori_loop(0, ITERS, body, 0)
    plsc.subcore_barrier()
    pltpu.sync_copy(
        acc_sh.at[pl.ds(base, ROWS_PER_TILE)],
        out_hbm.at[c].at[pl.ds(base, ROWS_PER_TILE)],
    )


@functools.partial(
    pl.kernel,
    mesh=_mesh,
    compiler_params=pltpu.CompilerParams(use_tc_tiling_on_sc=False),
    out_type=jax.ShapeDtypeStruct((NC, N_PAD, DEGW), jnp.float32),
    scratch_types=[
        pltpu.VMEM((ITERS, CH), jnp.int32),
        pltpu.VMEM((ITERS, CH), jnp.int32),
        pltpu.VMEM((CH, DEGW), jnp.float32),
        pltpu.VMEM_SHARED((N_PAD, DEGW), jnp.float32),
        pltpu.SemaphoreType.DMA,
    ],
)
def _sc_deg(table_hbm, src_hbm, dst_hbm, zeros_hbm, out_hbm,
            src_v, dst_v, rows_a, acc_sh, sem):
    c = lax.axis_index("c")
    s = lax.axis_index("s")
    wid = s * NC + c
    base = s * ROWS_PER_TILE
    pltpu.sync_copy(zeros_hbm, acc_sh.at[pl.ds(base, ROWS_PER_TILE)])
    pltpu.sync_copy(src_hbm.at[wid], src_v)
    pltpu.sync_copy(dst_hbm.at[wid], dst_v)
    plsc.subcore_barrier()

    def body(j, carry):
        pltpu.async_copy(table_hbm.at[src_v.at[j]], rows_a, sem).wait()
        pltpu.sync_copy(rows_a, acc_sh.at[dst_v.at[j]], add=True)
        return carry

    lax.fori_loop(0, ITERS, body, 0)
    plsc.subcore_barrier()
    pltpu.sync_copy(
        acc_sh.at[pl.ds(base, ROWS_PER_TILE)],
        out_hbm.at[c].at[pl.ds(base, ROWS_PER_TILE)],
    )


_BLK = 1024


def _tc_mm_scale(x_pad, w_pad, degp):
    def body(x_ref, w_ref, dp_ref, t0_ref, dinv_ref, dinv2_ref):
        y0 = jnp.dot(x_ref[...], w_ref[...], preferred_element_type=jnp.float32)
        dp = dp_ref[...]
        deg = dp[0, :, 0:1] + dp[1, :, 0:1] + 1.0
        dinv = lax.rsqrt(deg)
        dinv_ref[...] = dinv
        dinv2_ref[...] = 1.0 / deg
        t0_ref[...] = y0 * dinv

    grid = N_PAD // _BLK
    return pl.pallas_call(
        body,
        grid=(grid,),
        in_specs=[
            pl.BlockSpec((_BLK, D), lambda i: (i, 0)),
            pl.BlockSpec((D, WPAD), lambda i: (0, 0)),
            pl.BlockSpec((NC, _BLK, DEGW), lambda i: (0, i, 0)),
        ],
        out_specs=[
            pl.BlockSpec((_BLK, WPAD), lambda i: (i, 0)),
            pl.BlockSpec((_BLK, 1), lambda i: (i, 0)),
            pl.BlockSpec((_BLK, 1), lambda i: (i, 0)),
        ],
        out_shape=[
            jax.ShapeDtypeStruct((N_PAD, WPAD), jnp.float32),
            jax.ShapeDtypeStruct((N_PAD, 1), jnp.float32),
            jax.ShapeDtypeStruct((N_PAD, 1), jnp.float32),
        ],
    )(x_pad, w_pad, degp)


def _tc_combine(up, t_prev, scale):
    def body(up_ref, t_ref, s_ref, o_ref):
        u = up_ref[...]
        o_ref[...] = (u[0] + u[1] + t_ref[...]) * s_ref[...]

    grid = N_PAD // _BLK
    return pl.pallas_call(
        body,
        grid=(grid,),
        in_specs=[
            pl.BlockSpec((NC, _BLK, WPAD), lambda i: (0, i, 0)),
            pl.BlockSpec((_BLK, WPAD), lambda i: (i, 0)),
            pl.BlockSpec((_BLK, 1), lambda i: (i, 0)),
        ],
        out_specs=pl.BlockSpec((_BLK, WPAD), lambda i: (i, 0)),
        out_shape=jax.ShapeDtypeStruct((N_PAD, WPAD), jnp.float32),
    )(up, t_prev, scale)


def kernel(x, edge_index, W):
    src = edge_index[0]
    dst = edge_index[1]
    pad = E_PAD - E
    src_r = jnp.concatenate([src, jnp.zeros((pad,), jnp.int32)]).reshape(NW, ITERS, CH)
    dst_r = jnp.concatenate([dst, jnp.full((pad,), N, jnp.int32)]).reshape(NW, ITERS, CH)

    x_pad = jnp.pad(x, ((0, N_PAD - N), (0, 0)))
    w_pad = jnp.pad(W, ((0, 0), (0, WPAD - C)))

    zeros_hop = jnp.zeros((ROWS_PER_TILE, WPAD), jnp.float32)
    zeros_deg = jnp.zeros((ROWS_PER_TILE, DEGW), jnp.float32)
    ones_table = jnp.ones((N_PAD, DEGW), jnp.float32)

    degp = _sc_deg(ones_table, src_r, dst_r, zeros_deg)
    t0, dinv, dinv2 = _tc_mm_scale(x_pad, w_pad, degp)
    u1p = _sc_hop(t0, src_r, dst_r, zeros_hop)
    t1 = _tc_combine(u1p, t0, dinv2)
    u2p = _sc_hop(t1, src_r, dst_r, zeros_hop)
    out = _tc_combine(u2p, t1, dinv)
    return out[:N, :C]
